# baseline (speedup 1.0000x reference)
import functools

import jax
import jax.numpy as jnp
from jax import lax
from jax.experimental import pallas as pl
from jax.experimental.pallas import tpu as pltpu
from jax.experimental.pallas import tpu_sc as plsc

B, F, D, V = 4096, 26, 64, 100000
NC, NS = 2, 16
NW = NC * NS
BPW = B // NW
BB = 1024
FA = 16


def _sc_gather(item_idx, table_t, nf):
    mesh = plsc.VectorSubcoreMesh(
        core_axis_name="c", subcore_axis_name="s", num_cores=NC, num_subcores=NS
    )

    @functools.partial(
        pl.kernel,
        out_type=jax.ShapeDtypeStruct((nf, B), jnp.float32),
        mesh=mesh,
        scratch_types=[
            pltpu.VMEM((BPW,), jnp.int32),
            pltpu.VMEM((nf, BPW), jnp.float32),
            pltpu.SemaphoreType.DMA,
        ],
        compiler_params=pltpu.CompilerParams(
            needs_layout_passes=False,
            skip_device_barrier=True,
            use_tc_tiling_on_sc=False,
        ),
    )
    def gather_kernel(idx_hbm, table_hbm, out_hbm, idx_v, rows_v, sem):
        wid = lax.axis_index("s") * NC + lax.axis_index("c")
        base = wid * BPW
        pltpu.sync_copy(idx_hbm.at[pl.ds(base, BPW)], idx_v)
        for f in range(nf):
            pltpu.async_copy(
                table_hbm.at[f].at[idx_v], rows_v.at[f], sem
            )
        for f in range(nf):
            pltpu.make_async_copy(
                table_hbm.at[f].at[idx_v], rows_v.at[f], sem
            ).wait()
        pltpu.sync_copy(rows_v, out_hbm.at[:, pl.ds(base, BPW)])

    return gather_kernel(item_idx, table_t)


def _pool_body(a_ref, b_ref, x_ref, out_ref):
    a = jnp.concatenate([a_ref[...], b_ref[...]], axis=0)
    m = jnp.max(a, axis=0, keepdims=True)
    e = jnp.exp(a - m)
    w = e / jnp.sum(e, axis=0, keepdims=True)
    acc = x_ref[0] * w[0:1, :]
    for f in range(1, F):
        acc = acc + x_ref[f] * w[f : f + 1, :]
    out_ref[...] = acc


def _tc_pool(alpha_a, alpha_b, stack_t):
    return pl.pallas_call(
        _pool_body,
        grid=(B // BB,),
        in_specs=[
            pl.BlockSpec((FA, BB), lambda i: (0, i)),
            pl.BlockSpec((F - FA, BB), lambda i: (0, i)),
            pl.BlockSpec((F, D, BB), lambda i: (0, 0, i)),
        ],
        out_specs=pl.BlockSpec((D, BB), lambda i: (0, i)),
        out_shape=jax.ShapeDtypeStruct((D, B), jnp.float32),
        compiler_params=pltpu.CompilerParams(skip_device_barrier=True),
    )(alpha_a, alpha_b, stack_t)


def kernel(stack_embedding, item_input, alpha_embeddings):
    item_idx = jnp.reshape(item_input, (B,)).astype(jnp.int32)
    table_t = jnp.transpose(alpha_embeddings)
    stack_t = jnp.transpose(stack_embedding, (1, 2, 0))
    alpha_a = _sc_gather(item_idx, table_t[:FA], FA)
    alpha_b = _sc_gather(item_idx, table_t[FA:], F - FA)
    out_t = _tc_pool(alpha_a, alpha_b, stack_t)
    return jnp.transpose(out_t)


# speedup vs baseline: 1.2020x; 1.2020x over previous
import functools

import jax
import jax.numpy as jnp
from jax import lax
from jax.experimental import pallas as pl
from jax.experimental.pallas import tpu as pltpu
from jax.experimental.pallas import tpu_sc as plsc

B, F, D, V = 4096, 26, 64, 100000
NC, NS = 2, 16
NW = NC * NS
BPW = B // NW
BB = 1024


def _sc_gather(item_idx, table_t):
    mesh = plsc.VectorSubcoreMesh(
        core_axis_name="c", subcore_axis_name="s", num_cores=NC, num_subcores=NS
    )

    @functools.partial(
        pl.kernel,
        out_type=jax.ShapeDtypeStruct((F, B), jnp.float32),
        mesh=mesh,
        scratch_types=[
            pltpu.VMEM((BPW,), jnp.int32),
            pltpu.VMEM((F, BPW), jnp.float32),
            pltpu.SemaphoreType.DMA,
        ],
        compiler_params=pltpu.CompilerParams(
            needs_layout_passes=False,
            skip_device_barrier=True,
            use_tc_tiling_on_sc=False,
        ),
    )
    def gather_kernel(idx_hbm, table_hbm, out_hbm, idx_v, rows_v, sem):
        wid = lax.axis_index("s") * NC + lax.axis_index("c")
        base = wid * BPW
        pltpu.sync_copy(idx_hbm.at[pl.ds(base, BPW)], idx_v)
        for f in range(F):
            pltpu.async_copy(
                table_hbm.at[f].at[idx_v], rows_v.at[f], sem
            )
        for f in range(F):
            pltpu.make_async_copy(
                table_hbm.at[f].at[idx_v], rows_v.at[f], sem
            ).wait()
        pltpu.sync_copy(rows_v, out_hbm.at[:, pl.ds(base, BPW)])

    return gather_kernel(item_idx, table_t)


def _pool_body(a_ref, x_ref, out_ref):
    a = a_ref[...]
    m = jnp.max(a, axis=0, keepdims=True)
    e = jnp.exp(a - m)
    w = e / jnp.sum(e, axis=0, keepdims=True)
    acc = x_ref[0] * w[0:1, :]
    for f in range(1, F):
        acc = acc + x_ref[f] * w[f : f + 1, :]
    out_ref[...] = acc


def _tc_pool(alpha_t, stack_t):
    return pl.pallas_call(
        _pool_body,
        grid=(B // BB,),
        in_specs=[
            pl.BlockSpec((F, BB), lambda i: (0, i)),
            pl.BlockSpec((F, D, BB), lambda i: (0, 0, i)),
        ],
        out_specs=pl.BlockSpec((D, BB), lambda i: (0, i)),
        out_shape=jax.ShapeDtypeStruct((D, B), jnp.float32),
        compiler_params=pltpu.CompilerParams(skip_device_barrier=True),
    )(alpha_t, stack_t)


def kernel(stack_embedding, item_input, alpha_embeddings):
    item_idx = jnp.reshape(item_input, (B,)).astype(jnp.int32)
    table_t = jnp.transpose(alpha_embeddings)
    stack_t = jnp.transpose(stack_embedding, (1, 2, 0))
    alpha_t = _sc_gather(item_idx, table_t)
    out_t = _tc_pool(alpha_t, stack_t)
    return jnp.transpose(out_t)
